# Initial kernel scaffold; baseline (speedup 1.0000x reference)
import jax
import jax.numpy as jnp
from jax.experimental import pallas as pl


def kernel(weight, indices_fwd, offsets):
    raise NotImplementedError("write your pallas kernel here")


# speedup vs baseline: 118.6385x; 118.6385x over previous
import functools

import jax
import jax.numpy as jnp
from jax import lax
from jax.experimental import pallas as pl
from jax.experimental.pallas import tpu as pltpu
from jax.experimental.pallas import tpu_sc as plsc

N_TABLE = 100000
EMBED_DIM = 64
NUM_BAGS = 4096
TOTAL_INDICES = 204800

NC = 2
NS = 16
NW = NC * NS
CHUNK = 128
PER_W = TOTAL_INDICES // NW
N_CHUNKS = PER_W // CHUNK
LOG2_BAGS = 12


def _sc_partials():
    mesh = plsc.VectorSubcoreMesh(core_axis_name="c", subcore_axis_name="s")

    @functools.partial(
        pl.kernel,
        mesh=mesh,
        compiler_params=pltpu.CompilerParams(
            needs_layout_passes=False, use_tc_tiling_on_sc=False),
        out_type=jax.ShapeDtypeStruct((NC * NUM_BAGS, EMBED_DIM), jnp.float32),
        scratch_types=[
            pltpu.VMEM((NUM_BAGS,), jnp.int32),
            pltpu.VMEM((PER_W,), jnp.int32),
            pltpu.VMEM((CHUNK,), jnp.int32),
            pltpu.VMEM((CHUNK,), jnp.int32),
            pltpu.VMEM((CHUNK, EMBED_DIM), jnp.float32),
            pltpu.VMEM((CHUNK, EMBED_DIM), jnp.float32),
            pltpu.VMEM_SHARED((NUM_BAGS, EMBED_DIM), jnp.float32),
            pltpu.SemaphoreType.DMA,
            pltpu.SemaphoreType.DMA,
            pltpu.SemaphoreType.DMA,
            pltpu.SemaphoreType.DMA,
        ],
    )
    def body(weight_hbm, idx_hbm, offs_hbm, out_hbm, offs_v, idx_all,
             seg0, seg1, rows0, rows1, acc_sh, sem_g0, sem_g1, sem_s0,
             sem_s1):
        c = lax.axis_index("c")
        s = lax.axis_index("s")
        wid = s * NC + c

        pltpu.sync_copy(offs_hbm, offs_v)
        pltpu.sync_copy(idx_hbm.at[pl.ds(wid * PER_W, PER_W)], idx_all)

        zeros16 = jnp.zeros((16,), jnp.float32)

        def zero_row(r, _):
            for d in range(EMBED_DIM // 16):
                rows0[r, pl.ds(d * 16, 16)] = zeros16
            return _

        lax.fori_loop(0, CHUNK, zero_row, None)

        rows_per_tile = NUM_BAGS // NS
        for h in range(rows_per_tile // CHUNK):
            pltpu.sync_copy(
                rows0, acc_sh.at[pl.ds(s * rows_per_tile + h * CHUNK, CHUNK)])
        plsc.subcore_barrier()

        bufs = ((seg0, rows0, sem_g0, sem_s0), (seg1, rows1, sem_g1, sem_s1))

        def idx_slice(g):
            return idx_all.at[pl.ds(g * CHUNK, CHUNK)]

        def compute_segs(g, seg_ref):
            base = wid * PER_W + g * CHUNK
            for v in range(CHUNK // 16):
                pos = base + v * 16 + lax.broadcasted_iota(jnp.int32, (16,), 0)
                lo = jnp.zeros((16,), jnp.int32)
                hi = jnp.full((16,), NUM_BAGS, jnp.int32)
                for _step in range(LOG2_BAGS):
                    mid = (lo + hi) >> 1
                    val = plsc.load_gather(offs_v, [mid])
                    go_right = val <= pos
                    lo = jnp.where(go_right, mid + 1, lo)
                    hi = jnp.where(go_right, hi, mid)
                seg = lo - 1
                seg = jnp.maximum(seg, 0)
                seg = jnp.minimum(seg, NUM_BAGS - 1)
                seg_ref[pl.ds(v * 16, 16)] = seg

        def fire_gather(g, b):
            _, rows, semg, _ = bufs[b]
            pltpu.async_copy(weight_hbm.at[idx_slice(g)], rows, semg)

        def wait_gather(g, b):
            _, rows, semg, _ = bufs[b]
            pltpu.make_async_copy(
                weight_hbm.at[idx_slice(g)], rows, semg).wait()

        def fire_scatter(b):
            seg, rows, _, sems = bufs[b]
            pltpu.async_copy(rows, acc_sh.at[seg], sems, add=True)

        def wait_scatter(b):
            seg, rows, _, sems = bufs[b]
            pltpu.make_async_copy(rows, acc_sh.at[seg], sems).wait()

        def stage(g, b, fire_next, first):
            wait_gather(g, b)
            compute_segs(g, bufs[b][0])
            if not first:
                wait_scatter(1 - b)
            if fire_next:
                fire_gather(g + 1, 1 - b)
            fire_scatter(b)

        fire_gather(0, 0)
        stage(0, 0, True, True)

        def pair(i, _):
            g = 2 * i + 1
            stage(g, 1, True, False)
            stage(g + 1, 0, True, False)
            return _

        lax.fori_loop(0, (N_CHUNKS - 2) // 2, pair, None)
        stage(N_CHUNKS - 1, 1, False, False)
        wait_scatter(1)
        plsc.subcore_barrier()

        pltpu.sync_copy(
            acc_sh.at[pl.ds(s * rows_per_tile, rows_per_tile)],
            out_hbm.at[pl.ds(c * NUM_BAGS + s * rows_per_tile,
                             rows_per_tile)])

    return body


def _combine_body(a_ref, b_ref, o_ref):
    o_ref[...] = a_ref[...] + b_ref[...]


@jax.jit
def kernel(weight, indices_fwd, offsets):
    partials = _sc_partials()(weight, indices_fwd, offsets)
    return pl.pallas_call(
        _combine_body,
        out_shape=jax.ShapeDtypeStruct((NUM_BAGS, EMBED_DIM), jnp.float32),
    )(partials[:NUM_BAGS], partials[NUM_BAGS:])


# speedup vs baseline: 138.4789x; 1.1672x over previous
import functools

import jax
import jax.numpy as jnp
from jax import lax
from jax.experimental import pallas as pl
from jax.experimental.pallas import tpu as pltpu
from jax.experimental.pallas import tpu_sc as plsc

N_TABLE = 100000
EMBED_DIM = 64
NUM_BAGS = 4096
TOTAL_INDICES = 204800

NC = 2
NS = 16
NW = NC * NS
CHUNK = 128
PER_W = TOTAL_INDICES // NW
N_CHUNKS = PER_W // CHUNK
LOG2_BAGS = 12


def _sc_partials():
    mesh = plsc.VectorSubcoreMesh(core_axis_name="c", subcore_axis_name="s")

    @functools.partial(
        pl.kernel,
        mesh=mesh,
        compiler_params=pltpu.CompilerParams(
            needs_layout_passes=False, use_tc_tiling_on_sc=False),
        out_type=jax.ShapeDtypeStruct((NC * NUM_BAGS, EMBED_DIM), jnp.float32),
        scratch_types=[
            pltpu.VMEM((NUM_BAGS,), jnp.int32),
            pltpu.VMEM((PER_W,), jnp.int32),
            pltpu.VMEM((CHUNK,), jnp.int32),
            pltpu.VMEM((CHUNK,), jnp.int32),
            pltpu.VMEM((CHUNK, EMBED_DIM), jnp.float32),
            pltpu.VMEM((CHUNK, EMBED_DIM), jnp.float32),
            pltpu.VMEM_SHARED((NUM_BAGS, EMBED_DIM), jnp.float32),
            pltpu.SemaphoreType.DMA,
            pltpu.SemaphoreType.DMA,
            pltpu.SemaphoreType.DMA,
            pltpu.SemaphoreType.DMA,
        ],
    )
    def body(weight_hbm, idx_hbm, offs_hbm, out_hbm, offs_v, idx_all,
             seg0, seg1, rows0, rows1, acc_sh, sem_g0, sem_g1, sem_s0,
             sem_s1):
        c = lax.axis_index("c")
        s = lax.axis_index("s")
        wid = s * NC + c

        pltpu.sync_copy(offs_hbm, offs_v)
        pltpu.sync_copy(idx_hbm.at[pl.ds(wid * PER_W, PER_W)], idx_all)

        zeros16 = jnp.zeros((16,), jnp.float32)

        def zero_row(r, _):
            for d in range(EMBED_DIM // 16):
                rows0[r, pl.ds(d * 16, 16)] = zeros16
            return _

        lax.fori_loop(0, CHUNK, zero_row, None)

        rows_per_tile = NUM_BAGS // NS
        for h in range(rows_per_tile // CHUNK):
            pltpu.sync_copy(
                rows0, acc_sh.at[pl.ds(s * rows_per_tile + h * CHUNK, CHUNK)])
        plsc.subcore_barrier()

        bufs = ((seg0, rows0, sem_g0, sem_s0), (seg1, rows1, sem_g1, sem_s1))

        def idx_slice(g):
            return idx_all.at[pl.ds(g * CHUNK, CHUNK)]

        iota16 = lax.broadcasted_iota(jnp.int32, (16,), 0)
        n_vecs = CHUNK // 16

        def compute_segs(g, seg_ref):
            base = wid * PER_W + g * CHUNK
            pos0 = base + iota16
            lo = jnp.zeros((16,), jnp.int32)
            hi = jnp.full((16,), NUM_BAGS, jnp.int32)
            for _step in range(LOG2_BAGS):
                mid = (lo + hi) >> 1
                val = plsc.load_gather(offs_v, [mid])
                go_right = val <= pos0
                lo = jnp.where(go_right, mid + 1, lo)
                hi = jnp.where(go_right, hi, mid)
            s0 = jnp.maximum(lax.reduce_min(lo - 1, (0,)), 0)
            max_pos = base + CHUNK - 1

            def offs_at(k):
                kc = jnp.minimum(k, NUM_BAGS - 1)
                return lax.reduce_max(
                    plsc.load_gather(offs_v, [jnp.full((16,), kc, jnp.int32)]),
                    (0,))

            def w_cond(carry):
                k, vk = carry[0], carry[1]
                return (k < NUM_BAGS) & (vk <= max_pos)

            def w_body(carry):
                k, vk = carry[0], carry[1]
                cs = carry[2:]
                vkv = jnp.full((16,), vk, jnp.int32)
                cs = tuple(
                    cs[v] + jnp.where(vkv <= base + v * 16 + iota16, 1, 0)
                    for v in range(n_vecs))
                return (k + 1, offs_at(k + 1)) + cs

            init = (s0 + 1, offs_at(s0 + 1)) + tuple(
                jnp.zeros((16,), jnp.int32) for _ in range(n_vecs))
            out = lax.while_loop(w_cond, w_body, init)
            cs = out[2:]
            for v in range(n_vecs):
                seg = s0 + cs[v]
                seg = jnp.minimum(seg, NUM_BAGS - 1)
                seg_ref[pl.ds(v * 16, 16)] = seg

        def fire_gather(g, b):
            _, rows, semg, _ = bufs[b]
            pltpu.async_copy(weight_hbm.at[idx_slice(g)], rows, semg)

        def wait_gather(g, b):
            _, rows, semg, _ = bufs[b]
            pltpu.make_async_copy(
                weight_hbm.at[idx_slice(g)], rows, semg).wait()

        def fire_scatter(b):
            seg, rows, _, sems = bufs[b]
            pltpu.async_copy(rows, acc_sh.at[seg], sems, add=True)

        def wait_scatter(b):
            seg, rows, _, sems = bufs[b]
            pltpu.make_async_copy(rows, acc_sh.at[seg], sems).wait()

        def stage(g, b, fire_next, first):
            wait_gather(g, b)
            compute_segs(g, bufs[b][0])
            if not first:
                wait_scatter(1 - b)
            if fire_next:
                fire_gather(g + 1, 1 - b)
            fire_scatter(b)

        fire_gather(0, 0)
        stage(0, 0, True, True)

        def pair(i, _):
            g = 2 * i + 1
            stage(g, 1, True, False)
            stage(g + 1, 0, True, False)
            return _

        lax.fori_loop(0, (N_CHUNKS - 2) // 2, pair, None)
        stage(N_CHUNKS - 1, 1, False, False)
        wait_scatter(1)
        plsc.subcore_barrier()

        pltpu.sync_copy(
            acc_sh.at[pl.ds(s * rows_per_tile, rows_per_tile)],
            out_hbm.at[pl.ds(c * NUM_BAGS + s * rows_per_tile,
                             rows_per_tile)])

    return body


def _combine_body(a_ref, b_ref, o_ref):
    o_ref[...] = a_ref[...] + b_ref[...]


@jax.jit
def kernel(weight, indices_fwd, offsets):
    partials = _sc_partials()(weight, indices_fwd, offsets)
    return pl.pallas_call(
        _combine_body,
        out_shape=jax.ShapeDtypeStruct((NUM_BAGS, EMBED_DIM), jnp.float32),
    )(partials[:NUM_BAGS], partials[NUM_BAGS:])


# speedup vs baseline: 153.3722x; 1.1075x over previous
import functools

import jax
import jax.numpy as jnp
from jax import lax
from jax.experimental import pallas as pl
from jax.experimental.pallas import tpu as pltpu
from jax.experimental.pallas import tpu_sc as plsc

N_TABLE = 100000
EMBED_DIM = 64
NUM_BAGS = 4096
TOTAL_INDICES = 204800

NC = 2
NS = 16
NW = NC * NS
CHUNK = 256
SCAT = 128
PER_W = TOTAL_INDICES // NW
N_CHUNKS = PER_W // CHUNK
LOG2_BAGS = 12


def _sc_partials():
    mesh = plsc.VectorSubcoreMesh(core_axis_name="c", subcore_axis_name="s")

    @functools.partial(
        pl.kernel,
        mesh=mesh,
        compiler_params=pltpu.CompilerParams(
            needs_layout_passes=False, use_tc_tiling_on_sc=False),
        out_type=jax.ShapeDtypeStruct((NC * NUM_BAGS, EMBED_DIM), jnp.float32),
        scratch_types=[
            pltpu.VMEM((NUM_BAGS,), jnp.int32),
            pltpu.VMEM((PER_W,), jnp.int32),
            pltpu.VMEM((SCAT,), jnp.int32),
            pltpu.VMEM((SCAT,), jnp.int32),
            pltpu.VMEM((SCAT,), jnp.int32),
            pltpu.VMEM((SCAT,), jnp.int32),
            pltpu.VMEM((CHUNK, EMBED_DIM), jnp.float32),
            pltpu.VMEM((CHUNK, EMBED_DIM), jnp.float32),
            pltpu.VMEM_SHARED((NUM_BAGS, EMBED_DIM), jnp.float32),
            pltpu.SemaphoreType.DMA,
            pltpu.SemaphoreType.DMA,
            pltpu.SemaphoreType.DMA,
            pltpu.SemaphoreType.DMA,
        ],
    )
    def body(weight_hbm, idx_hbm, offs_hbm, out_hbm, offs_v, idx_all,
             seg0a, seg0b, seg1a, seg1b, rows0, rows1, acc_sh,
             sem_g0, sem_g1, sem_s0, sem_s1):
        c = lax.axis_index("c")
        s = lax.axis_index("s")
        wid = s * NC + c

        pltpu.sync_copy(offs_hbm, offs_v)
        pltpu.sync_copy(idx_hbm.at[pl.ds(wid * PER_W, PER_W)], idx_all)

        zeros16 = jnp.zeros((16,), jnp.float32)

        def zero_row(r, _):
            for d in range(EMBED_DIM // 16):
                rows0[r, pl.ds(d * 16, 16)] = zeros16
            return _

        lax.fori_loop(0, CHUNK, zero_row, None)

        rows_per_tile = NUM_BAGS // NS
        pltpu.sync_copy(rows0, acc_sh.at[pl.ds(s * rows_per_tile, CHUNK)])
        plsc.subcore_barrier()

        bufs = (((seg0a, seg0b), rows0, sem_g0, sem_s0),
                ((seg1a, seg1b), rows1, sem_g1, sem_s1))

        iota16 = lax.broadcasted_iota(jnp.int32, (16,), 0)
        n_vecs = CHUNK // 16

        def idx_slice(g):
            return idx_all.at[pl.ds(g * CHUNK, CHUNK)]

        def compute_segs(g, seg_refs):
            base = wid * PER_W + g * CHUNK
            pos0 = base + iota16
            lo = jnp.zeros((16,), jnp.int32)
            hi = jnp.full((16,), NUM_BAGS, jnp.int32)
            for _step in range(LOG2_BAGS):
                mid = (lo + hi) >> 1
                val = plsc.load_gather(offs_v, [mid])
                go_right = val <= pos0
                lo = jnp.where(go_right, mid + 1, lo)
                hi = jnp.where(go_right, hi, mid)
            s0 = jnp.maximum(lax.reduce_min(lo - 1, (0,)), 0)
            max_pos = base + CHUNK - 1

            def offs_at(k):
                kc = jnp.minimum(k, NUM_BAGS - 1)
                return lax.reduce_max(
                    plsc.load_gather(offs_v, [jnp.full((16,), kc, jnp.int32)]),
                    (0,))

            def w_cond(carry):
                k, vk = carry[0], carry[1]
                return (k < NUM_BAGS) & (vk <= max_pos)

            def w_body(carry):
                k, vk = carry[0], carry[1]
                cs = carry[2:]
                vkv = jnp.full((16,), vk, jnp.int32)
                cs = tuple(
                    cs[v] + jnp.where(vkv <= base + v * 16 + iota16, 1, 0)
                    for v in range(n_vecs))
                return (k + 1, offs_at(k + 1)) + cs

            init = (s0 + 1, offs_at(s0 + 1)) + tuple(
                jnp.zeros((16,), jnp.int32) for _ in range(n_vecs))
            out = lax.while_loop(w_cond, w_body, init)
            cs = out[2:]
            for v in range(n_vecs):
                seg = jnp.minimum(s0 + cs[v], NUM_BAGS - 1)
                half, off = divmod(v, SCAT // 16)
                seg_refs[half][pl.ds(off * 16, 16)] = seg

        def fire_gather(g, b):
            _, rows, semg, _ = bufs[b]
            pltpu.async_copy(weight_hbm.at[idx_slice(g)], rows, semg)

        def wait_gather(g, b):
            _, rows, semg, _ = bufs[b]
            pltpu.make_async_copy(
                weight_hbm.at[idx_slice(g)], rows, semg).wait()

        def fire_scatter(b):
            segs, rows, _, sems = bufs[b]
            pltpu.async_copy(
                rows.at[pl.ds(0, SCAT)], acc_sh.at[segs[0]], sems, add=True)
            pltpu.async_copy(
                rows.at[pl.ds(SCAT, SCAT)], acc_sh.at[segs[1]], sems, add=True)

        def wait_scatter(b):
            segs, rows, _, sems = bufs[b]
            pltpu.make_async_copy(
                rows.at[pl.ds(0, SCAT)], acc_sh.at[segs[0]], sems).wait()
            pltpu.make_async_copy(
                rows.at[pl.ds(SCAT, SCAT)], acc_sh.at[segs[1]], sems).wait()

        def stage(g, b, fire_next, first):
            wait_gather(g, b)
            compute_segs(g, bufs[b][0])
            if not first:
                wait_scatter(1 - b)
            if fire_next:
                fire_gather(g + 1, 1 - b)
            fire_scatter(b)

        fire_gather(0, 0)
        stage(0, 0, True, True)
        stage(1, 1, True, False)

        def pair(i, _):
            g = 2 * i + 2
            stage(g, 0, True, False)
            stage(g + 1, 1, True, False)
            return _

        lax.fori_loop(0, (N_CHUNKS - 3) // 2, pair, None)
        stage(N_CHUNKS - 1, 0, False, False)
        wait_scatter(0)
        plsc.subcore_barrier()

        pltpu.sync_copy(
            acc_sh.at[pl.ds(s * rows_per_tile, rows_per_tile)],
            out_hbm.at[pl.ds(c * NUM_BAGS + s * rows_per_tile,
                             rows_per_tile)])

    return body


def _combine_body(a_ref, b_ref, o_ref):
    o_ref[...] = a_ref[...] + b_ref[...]


@jax.jit
def kernel(weight, indices_fwd, offsets):
    partials = _sc_partials()(weight, indices_fwd, offsets)
    return pl.pallas_call(
        _combine_body,
        out_shape=jax.ShapeDtypeStruct((NUM_BAGS, EMBED_DIM), jnp.float32),
    )(partials[:NUM_BAGS], partials[NUM_BAGS:])
